# baseline (speedup 1.0000x reference)
import functools

import jax
import jax.numpy as jnp
from jax import lax
from jax.experimental import pallas as pl
from jax.experimental.pallas import tpu as pltpu
from jax.experimental.pallas import tpu_sc as plsc

_NUM_WORKERS = 32
_CHUNK = 16
_NBUF = 4


def _make_sc_gather(B, V, D):
    b_per_w = B // _NUM_WORKERS
    n_chunks = b_per_w // _CHUNK
    n_groups = n_chunks // _NBUF
    depth = _NBUF - 1
    mesh = plsc.VectorSubcoreMesh(core_axis_name="c", subcore_axis_name="s")

    @functools.partial(
        pl.kernel,
        mesh=mesh,
        out_type=jax.ShapeDtypeStruct((B, D), jnp.float32),
        scratch_types=[
            pltpu.VMEM((b_per_w,), jnp.int32),
            pltpu.VMEM((b_per_w,), jnp.int32),
        ]
        + [pltpu.VMEM((_CHUNK, D), jnp.float32)] * _NBUF
        + [pltpu.SemaphoreType.DMA] * _NBUF
        + [pltpu.SemaphoreType.DMA],
    )
    def gather_kernel(idx_hbm, pos_hbm, table_hbm, out_hbm, idx_v, pos_v,
                      *rest):
        bufs = rest[:_NBUF]
        sems = rest[_NBUF:2 * _NBUF]
        osem = rest[2 * _NBUF]
        wid = lax.axis_index("s") * 2 + lax.axis_index("c")
        base = wid * b_per_w
        pltpu.sync_copy(idx_hbm.at[pl.ds(base, b_per_w)], idx_v)
        pltpu.sync_copy(pos_hbm.at[pl.ds(base, b_per_w)], pos_v)

        def gather(c, j):
            off = pl.multiple_of(c * _CHUNK, 8)
            pltpu.async_copy(
                table_hbm.at[idx_v.at[pl.ds(off, _CHUNK)]], bufs[j], sems[j])

        def put(c, j):
            off = pl.multiple_of(c * _CHUNK, 8)
            pvec = pos_v[pl.ds(off, 16)]
            for k in range(_CHUNK):
                p = pvec[k]
                pltpu.async_copy(
                    bufs[j].at[pl.ds(k, 1)],
                    out_hbm.at[pl.ds(p, 1)], osem)
            for k in range(_CHUNK):
                pltpu.make_async_copy(
                    bufs[j].at[pl.ds(k, 1)],
                    out_hbm.at[pl.ds(base, 1)], osem).wait()

        def drain(j):
            pltpu.make_async_copy(
                table_hbm.at[pl.ds(0, _CHUNK)], bufs[j], sems[j]).wait()

        for j in range(depth):
            gather(j, j)

        def body(g, carry):
            c0 = g * _NBUF
            for j in range(_NBUF):
                c = c0 + j
                nxt = c + depth
                pl.when(nxt < n_chunks)(
                    lambda: gather(nxt, (j + depth) % _NBUF))
                drain(j)
                put(c, j)
            return carry

        lax.fori_loop(0, n_groups, body, 0)

    return gather_kernel


def kernel(x, pe):
    x_shape = x.shape
    V, D = pe.shape
    flat = x.reshape(-1)
    B = flat.shape[0]
    pos = jnp.arange(B, dtype=jnp.int32)
    out = _make_sc_gather(B, V, D)(flat, pos, pe)
    return out.reshape(x_shape + (D,))


# speedup vs baseline: 1.0668x; 1.0668x over previous
import functools

import jax
import jax.numpy as jnp
from jax import lax
from jax.experimental import pallas as pl
from jax.experimental.pallas import tpu as pltpu
from jax.experimental.pallas import tpu_sc as plsc

_NUM_WORKERS = 32
_CHUNK = 16
_NBUF = 4


def _make_sc_gather(B, V, D):
    b_per_w = B // _NUM_WORKERS
    n_chunks = b_per_w // _CHUNK
    n_groups = n_chunks // _NBUF
    depth = _NBUF - 1
    mesh = plsc.VectorSubcoreMesh(core_axis_name="c", subcore_axis_name="s")

    @functools.partial(
        pl.kernel,
        mesh=mesh,
        out_type=jax.ShapeDtypeStruct((B, D), jnp.float32),
        scratch_types=[
            pltpu.VMEM((b_per_w,), jnp.int32),
        ]
        + [pltpu.VMEM((_CHUNK, D), jnp.float32)] * _NBUF
        + [pltpu.SemaphoreType.DMA] * _NBUF
        + [
            pltpu.VMEM_SHARED((16, 2, _CHUNK, D), jnp.float32),
            pltpu.SemaphoreType.DMA,
            pltpu.SemaphoreType.DMA,
        ],
    )
    def gather_kernel(idx_hbm, table_hbm, out_hbm, idx_v, *rest):
        bufs = rest[:_NBUF]
        sems = rest[_NBUF:2 * _NBUF]
        shared = rest[2 * _NBUF]
        osem = rest[2 * _NBUF + 1:]
        cid = lax.axis_index("c")
        sid = lax.axis_index("s")
        wid = sid * 2 + cid
        base = wid * b_per_w
        pltpu.sync_copy(idx_hbm.at[pl.ds(base, b_per_w)], idx_v)

        def gather(c, j):
            off = pl.multiple_of(c * _CHUNK, 8)
            pltpu.async_copy(
                table_hbm.at[idx_v.at[pl.ds(off, _CHUNK)]], bufs[j], sems[j])

        def put(c, j):
            off = pl.multiple_of(c * _CHUNK, 8)
            s = j % 2
            pl.when(c >= 2)(lambda: drain_o(s))
            pltpu.sync_copy(bufs[j], shared.at[sid, s])
            pltpu.async_copy(
                shared.at[sid, s],
                out_hbm.at[pl.ds(base + off, _CHUNK)], osem[s])

        def drain(j):
            pltpu.make_async_copy(
                table_hbm.at[pl.ds(0, _CHUNK)], bufs[j], sems[j]).wait()

        def drain_o(s):
            pltpu.make_async_copy(
                shared.at[sid, s],
                out_hbm.at[pl.ds(base, _CHUNK)], osem[s]).wait()

        for j in range(depth):
            gather(j, j)

        def body(g, carry):
            c0 = g * _NBUF
            for j in range(_NBUF):
                c = c0 + j
                nxt = c + depth
                pl.when(nxt < n_chunks)(
                    lambda: gather(nxt, (j + depth) % _NBUF))
                drain(j)
                put(c, j)
            return carry

        lax.fori_loop(0, n_groups, body, 0)
        drain_o(0)
        drain_o(1)

    return gather_kernel


def kernel(x, pe):
    x_shape = x.shape
    V, D = pe.shape
    flat = x.reshape(-1)
    B = flat.shape[0]
    out = _make_sc_gather(B, V, D)(flat, pe)
    return out.reshape(x_shape + (D,))
